# Initial kernel scaffold; baseline (speedup 1.0000x reference)
import jax
import jax.numpy as jnp
from jax.experimental import pallas as pl


def kernel(node_features, batch):
    raise NotImplementedError("write your pallas kernel here")


# speedup vs baseline: 6.1305x; 6.1305x over previous
import functools

import jax
import jax.numpy as jnp
from jax import lax
from jax.experimental import pallas as pl
from jax.experimental.pallas import tpu as pltpu
from jax.experimental.pallas import tpu_sc as plsc

N = 320000
D = 128
S = 1024
NC = 2
NS = 16
NW = NC * NS
R = N // NW
C = 40
CH = R // C
HALF = CH // 2
SS = S // NS


def _sc_partials(node_features, batch):
    mesh = plsc.VectorSubcoreMesh(core_axis_name="c", subcore_axis_name="s")

    @functools.partial(
        pl.kernel,
        out_type=jax.ShapeDtypeStruct((NC, S, D), jnp.float32),
        mesh=mesh,
        scratch_types=[
            pltpu.VMEM((C, D), jnp.float32),
            pltpu.VMEM((C, D), jnp.float32),
            pltpu.VMEM((C,), jnp.int32),
            pltpu.VMEM((C,), jnp.int32),
            pltpu.VMEM((SS, D), jnp.float32),
            pltpu.VMEM_SHARED((S, D), jnp.float32),
            pltpu.SemaphoreType.DMA,
            pltpu.SemaphoreType.DMA,
            pltpu.SemaphoreType.DMA,
            pltpu.SemaphoreType.DMA,
        ],
    )
    def k(nf_hbm, b_hbm, out_hbm, rows_a, rows_b, idx_a, idx_b, stage, acc,
          sem_ra, sem_rb, sem_ia, sem_ib):
        c = lax.axis_index("c")
        s = lax.axis_index("s")
        wid = s * NC + c
        base = wid * R

        zero = jnp.zeros((16,), jnp.float32)

        def zrow(i, carry):
            for j in range(D // 16):
                stage[i, pl.ds(j * 16, 16)] = zero
            return carry

        lax.fori_loop(0, SS, zrow, 0)
        pltpu.sync_copy(stage, acc.at[pl.ds(s * SS, SS)])
        plsc.subcore_barrier()

        pltpu.async_copy(nf_hbm.at[pl.ds(base, C)], rows_a, sem_ra)
        pltpu.async_copy(b_hbm.at[pl.ds(base, C)], idx_a, sem_ia)
        pltpu.async_copy(nf_hbm.at[pl.ds(base + C, C)], rows_b, sem_rb)
        pltpu.async_copy(b_hbm.at[pl.ds(base + C, C)], idx_b, sem_ib)

        def body(kk, carry):
            pltpu.make_async_copy(nf_hbm.at[pl.ds(base, C)], rows_a, sem_ra).wait()
            pltpu.make_async_copy(b_hbm.at[pl.ds(base, C)], idx_a, sem_ia).wait()
            pltpu.sync_copy(rows_a, acc.at[idx_a], add=True)

            @pl.when(kk < HALF - 1)
            def _():
                off = base + (2 * kk + 2) * C
                pltpu.async_copy(nf_hbm.at[pl.ds(off, C)], rows_a, sem_ra)
                pltpu.async_copy(b_hbm.at[pl.ds(off, C)], idx_a, sem_ia)

            pltpu.make_async_copy(nf_hbm.at[pl.ds(base, C)], rows_b, sem_rb).wait()
            pltpu.make_async_copy(b_hbm.at[pl.ds(base, C)], idx_b, sem_ib).wait()
            pltpu.sync_copy(rows_b, acc.at[idx_b], add=True)

            @pl.when(kk < HALF - 1)
            def _():
                off = base + (2 * kk + 3) * C
                pltpu.async_copy(nf_hbm.at[pl.ds(off, C)], rows_b, sem_rb)
                pltpu.async_copy(b_hbm.at[pl.ds(off, C)], idx_b, sem_ib)

            return carry

        lax.fori_loop(0, HALF, body, 0)

        plsc.subcore_barrier()
        pltpu.sync_copy(acc.at[pl.ds(s * SS, SS)], stage)
        pltpu.sync_copy(stage, out_hbm.at[c, pl.ds(s * SS, SS)])

    return k(node_features, batch)


def _merge(partials):
    def body(p_ref, o_ref):
        o_ref[...] = p_ref[0] + p_ref[1]

    return pl.pallas_call(
        body,
        out_shape=jax.ShapeDtypeStruct((S, D), jnp.float32),
    )(partials)


def kernel(node_features, batch):
    return _merge(_sc_partials(node_features, batch))
